# Initial kernel scaffold; baseline (speedup 1.0000x reference)
import jax
import jax.numpy as jnp
from jax.experimental import pallas as pl


def kernel(element_list, encodings):
    raise NotImplementedError("write your pallas kernel here")


# speedup vs baseline: 4.6914x; 4.6914x over previous
import functools

import jax
import jax.numpy as jnp
from jax import lax
from jax.experimental import pallas as pl
from jax.experimental.pallas import tpu as pltpu
from jax.experimental.pallas import tpu_sc as plsc

FEAT = 32
CHUNK = 128


def kernel(element_list, encodings):
    idx = element_list.reshape(-1).astype(jnp.int32)
    B = idx.shape[0]

    info = plsc.get_sparse_core_info()
    nw = info.num_cores * info.num_subcores
    b_per_w = B // nw
    n_chunks = b_per_w // CHUNK
    idx2 = idx.reshape(B // CHUNK, CHUNK)

    mesh = plsc.VectorSubcoreMesh(core_axis_name="c", subcore_axis_name="s")

    @functools.partial(
        pl.kernel,
        mesh=mesh,
        out_type=jax.ShapeDtypeStruct((B, FEAT), jnp.float32),
        scratch_types=[
            pltpu.VMEM((n_chunks, CHUNK), jnp.int32),
            pltpu.VMEM((CHUNK, FEAT), jnp.float32),
            pltpu.SemaphoreType.DMA,
        ],
        compiler_params=pltpu.CompilerParams(use_tc_tiling_on_sc=False),
    )
    def gather_kernel(idx_hbm, table_hbm, out_hbm, idx_v, rows_v, sem):
        wid = lax.axis_index("s") * info.num_cores + lax.axis_index("c")
        pltpu.sync_copy(idx_hbm.at[pl.ds(wid * n_chunks, n_chunks)], idx_v)

        def body(j, carry):
            pltpu.async_copy(table_hbm.at[idx_v.at[j]], rows_v, sem).wait()
            out_base = wid * b_per_w + j * CHUNK
            pltpu.sync_copy(rows_v, out_hbm.at[pl.ds(out_base, CHUNK)])
            return carry

        lax.fori_loop(0, n_chunks, body, 0)

    return gather_kernel(idx2, encodings)


# speedup vs baseline: 5.8543x; 1.2479x over previous
import functools

import jax
import jax.numpy as jnp
from jax import lax
from jax.experimental import pallas as pl
from jax.experimental.pallas import tpu as pltpu
from jax.experimental.pallas import tpu_sc as plsc

FEAT = 32
CHUNK = 128
K = 10
MEGA = K * CHUNK


def kernel(element_list, encodings):
    idx = element_list.reshape(-1).astype(jnp.int32)
    B = idx.shape[0]

    info = plsc.get_sparse_core_info()
    nw = info.num_cores * info.num_subcores
    b_per_w = B // nw
    n_chunks = b_per_w // CHUNK
    n_mega = b_per_w // MEGA
    idx2 = idx.reshape(B // CHUNK, CHUNK)

    mesh = plsc.VectorSubcoreMesh(core_axis_name="c", subcore_axis_name="s")

    @functools.partial(
        pl.kernel,
        mesh=mesh,
        out_type=jax.ShapeDtypeStruct((B, FEAT), jnp.float32),
        scratch_types=[
            pltpu.VMEM((n_chunks, CHUNK), jnp.int32),
            pltpu.VMEM((2, MEGA, FEAT), jnp.float32),
            pltpu.SemaphoreType.DMA,
            pltpu.SemaphoreType.DMA,
            pltpu.SemaphoreType.DMA,
        ],
        compiler_params=pltpu.CompilerParams(use_tc_tiling_on_sc=False),
    )
    def gather_kernel(idx_hbm, table_hbm, out_hbm, idx_v, rows_v, sem_g,
                      sem_o0, sem_o1):
        wid = lax.axis_index("s") * info.num_cores + lax.axis_index("c")
        out_base = wid * b_per_w
        sem_o = (sem_o0, sem_o1)
        pltpu.sync_copy(idx_hbm.at[pl.ds(wid * n_chunks, n_chunks)], idx_v)

        def fill(m, b):
            handles = []
            for k in range(K):
                handles.append(pltpu.async_copy(
                    table_hbm.at[idx_v.at[m * K + k]],
                    rows_v.at[b, pl.ds(k * CHUNK, CHUNK)],
                    sem_g))
            for h in handles:
                h.wait()

        def drain_out(b):
            pltpu.make_async_copy(
                rows_v.at[b], out_hbm.at[pl.ds(out_base, MEGA)],
                sem_o[b]).wait()

        def start_out(m, b):
            pltpu.async_copy(rows_v.at[b],
                             out_hbm.at[pl.ds(out_base + m * MEGA, MEGA)],
                             sem_o[b])

        for b in range(2):
            fill(b, b)
            start_out(b, b)

        @pl.loop(2, n_mega, step=2)
        def body(t):
            for b in range(2):
                m = t + b
                drain_out(b)
                fill(m, b)
                start_out(m, b)

        drain_out(0)
        drain_out(1)

    return gather_kernel(idx2, encodings)


# speedup vs baseline: 6.6347x; 1.1333x over previous
import functools

import jax
import jax.numpy as jnp
from jax import lax
from jax.experimental import pallas as pl
from jax.experimental.pallas import tpu as pltpu
from jax.experimental.pallas import tpu_sc as plsc

FEAT = 32
VOCAB = 100000
CH = 4096


def kernel(element_list, encodings):
    idx = element_list.reshape(-1).astype(jnp.int32)
    B = idx.shape[0]
    table_t = encodings.T

    info = plsc.get_sparse_core_info()
    nc, ns = info.num_cores, info.num_subcores
    nch = B // CH

    mesh = plsc.VectorSubcoreMesh(core_axis_name="c", subcore_axis_name="s")

    @functools.partial(
        pl.kernel,
        mesh=mesh,
        out_type=jax.ShapeDtypeStruct((FEAT, B), jnp.float32),
        scratch_types=[
            pltpu.VMEM((VOCAB,), jnp.float32),
            pltpu.VMEM((CH,), jnp.int32),
            pltpu.VMEM((CH,), jnp.int32),
            pltpu.VMEM((CH,), jnp.float32),
            pltpu.VMEM((CH,), jnp.float32),
            pltpu.SemaphoreType.DMA,
            pltpu.SemaphoreType.DMA,
            pltpu.SemaphoreType.DMA,
            pltpu.SemaphoreType.DMA,
        ],
        compiler_params=pltpu.CompilerParams(needs_layout_passes=False),
    )
    def gather_kernel(idx_hbm, table_hbm, out_hbm, tab_v, idx_v0,
                      idx_v1, out_v0, out_v1, sem_i0, sem_i1, sem_o0,
                      sem_o1):
        c = lax.axis_index("c")
        s = lax.axis_index("s")
        f = c * ns + s
        sem_i = (sem_i0, sem_i1)
        sem_o = (sem_o0, sem_o1)
        idx_v = (idx_v0, idx_v1)
        out_v = (out_v0, out_v1)

        pltpu.sync_copy(table_hbm.at[f], tab_v)

        off = lax.rem(f * 6, nch)

        def chunk_of(m):
            return lax.rem(m + off, nch)

        def start_idx(m, b):
            pltpu.async_copy(idx_hbm.at[pl.ds(chunk_of(m) * CH, CH)],
                             idx_v[b], sem_i[b])

        def wait_idx(b):
            pltpu.make_async_copy(idx_hbm.at[pl.ds(0, CH)], idx_v[b],
                                  sem_i[b]).wait()

        def compute(b):
            ib = idx_v[b]
            ob = out_v[b]

            @pl.loop(0, CH // 16, unroll=8)
            def _(j):
                iv = ib[pl.ds(j * 16, 16)]
                ob[pl.ds(j * 16, 16)] = plsc.load_gather(tab_v, [iv])

        def start_out(m, b):
            pltpu.async_copy(out_v[b],
                             out_hbm.at[f, pl.ds(chunk_of(m) * CH, CH)],
                             sem_o[b])

        def wait_out(b):
            pltpu.make_async_copy(out_v[b],
                                  out_hbm.at[f, pl.ds(0, CH)],
                                  sem_o[b]).wait()

        for b in range(2):
            start_idx(b, b)
        for b in range(2):
            wait_idx(b)
            compute(b)
            start_out(b, b)
            start_idx(b + 2, b)

        @pl.loop(2, nch - 2, step=2)
        def _(t):
            for b in range(2):
                m = t + b
                wait_idx(b)
                wait_out(b)
                compute(b)
                start_out(m, b)
                start_idx(m + 2, b)

        for b in range(2):
            wait_idx(b)
            wait_out(b)
            compute(b)
            start_out(nch - 2 + b, b)
        for b in range(2):
            wait_out(b)

    out_t = gather_kernel(idx, table_t)
    return out_t.T


# speedup vs baseline: 18.5495x; 2.7959x over previous
import functools

import jax
import jax.numpy as jnp
from jax import lax
from jax.experimental import pallas as pl
from jax.experimental.pallas import tpu as pltpu
from jax.experimental.pallas import tpu_sc as plsc

FEAT = 32
VOCAB = 100000
CH = 4096


def kernel(element_list, encodings):
    idx = element_list.reshape(-1).astype(jnp.int32)
    B = idx.shape[0]
    table_t = encodings.T

    info = plsc.get_sparse_core_info()
    nc, ns = info.num_cores, info.num_subcores
    nch = B // CH

    mesh = plsc.VectorSubcoreMesh(core_axis_name="c", subcore_axis_name="s")

    @functools.partial(
        pl.kernel,
        mesh=mesh,
        out_type=jax.ShapeDtypeStruct((FEAT, B), jnp.float32),
        scratch_types=[
            pltpu.VMEM((VOCAB,), jnp.float32),
            pltpu.VMEM((CH,), jnp.int32),
            pltpu.VMEM((CH,), jnp.int32),
            pltpu.VMEM((CH,), jnp.float32),
            pltpu.VMEM((CH,), jnp.float32),
            pltpu.SemaphoreType.DMA,
            pltpu.SemaphoreType.DMA,
            pltpu.SemaphoreType.DMA,
            pltpu.SemaphoreType.DMA,
        ],
        compiler_params=pltpu.CompilerParams(needs_layout_passes=False),
    )
    def gather_kernel(idx_hbm, table_hbm, out_hbm, tab_v, idx_v0,
                      idx_v1, out_v0, out_v1, sem_i0, sem_i1, sem_o0,
                      sem_o1):
        c = lax.axis_index("c")
        s = lax.axis_index("s")
        f = c * ns + s
        sem_i = (sem_i0, sem_i1)
        sem_o = (sem_o0, sem_o1)
        idx_v = (idx_v0, idx_v1)
        out_v = (out_v0, out_v1)

        pltpu.sync_copy(table_hbm.at[f], tab_v)

        off = lax.rem(f * 6, nch)

        def chunk_of(m):
            return lax.rem(m + off, nch)

        def start_idx(m, b):
            pltpu.async_copy(idx_hbm.at[pl.ds(chunk_of(m) * CH, CH)],
                             idx_v[b], sem_i[b])

        def wait_idx(b):
            pltpu.make_async_copy(idx_hbm.at[pl.ds(0, CH)], idx_v[b],
                                  sem_i[b]).wait()

        def compute(b):
            ib = idx_v[b]
            ob = out_v[b]

            @plsc.parallel_loop(0, CH, step=16, unroll=8)
            def _(j):
                iv = ib[pl.ds(j, 16)]
                ob[pl.ds(j, 16)] = plsc.load_gather(tab_v, [iv])

        def start_out(m, b):
            pltpu.async_copy(out_v[b],
                             out_hbm.at[f, pl.ds(chunk_of(m) * CH, CH)],
                             sem_o[b])

        def wait_out(b):
            pltpu.make_async_copy(out_v[b],
                                  out_hbm.at[f, pl.ds(0, CH)],
                                  sem_o[b]).wait()

        for b in range(2):
            start_idx(b, b)
        for b in range(2):
            wait_idx(b)
            compute(b)
            start_out(b, b)
            start_idx(b + 2, b)

        @pl.loop(2, nch - 2, step=2)
        def _(t):
            for b in range(2):
                m = t + b
                wait_idx(b)
                wait_out(b)
                compute(b)
                start_out(m, b)
                start_idx(m + 2, b)

        for b in range(2):
            wait_idx(b)
            wait_out(b)
            compute(b)
            start_out(nch - 2 + b, b)
        for b in range(2):
            wait_out(b)

    out_t = gather_kernel(idx, table_t)
    return out_t.T


# speedup vs baseline: 18.5590x; 1.0005x over previous
import functools

import jax
import jax.numpy as jnp
from jax import lax
from jax.experimental import pallas as pl
from jax.experimental.pallas import tpu as pltpu
from jax.experimental.pallas import tpu_sc as plsc

FEAT = 32
VOCAB = 100000
CH = 4096


def kernel(element_list, encodings):
    idx = element_list.reshape(-1).astype(jnp.int32)
    B = idx.shape[0]
    table_t = encodings.T

    info = plsc.get_sparse_core_info()
    nc, ns = info.num_cores, info.num_subcores
    nch = B // CH

    mesh = plsc.VectorSubcoreMesh(core_axis_name="c", subcore_axis_name="s")

    @functools.partial(
        pl.kernel,
        mesh=mesh,
        out_type=jax.ShapeDtypeStruct((FEAT, B), jnp.float32),
        scratch_types=[
            pltpu.VMEM((VOCAB,), jnp.float32),
            pltpu.VMEM((CH,), jnp.int32),
            pltpu.VMEM((CH,), jnp.int32),
            pltpu.VMEM((CH,), jnp.float32),
            pltpu.VMEM((CH,), jnp.float32),
            pltpu.SemaphoreType.DMA,
            pltpu.SemaphoreType.DMA,
            pltpu.SemaphoreType.DMA,
            pltpu.SemaphoreType.DMA,
        ],
        compiler_params=pltpu.CompilerParams(needs_layout_passes=False),
    )
    def gather_kernel(idx_hbm, table_hbm, out_hbm, tab_v, idx_v0,
                      idx_v1, out_v0, out_v1, sem_i0, sem_i1, sem_o0,
                      sem_o1):
        c = lax.axis_index("c")
        s = lax.axis_index("s")
        f = c * ns + s
        sem_i = (sem_i0, sem_i1)
        sem_o = (sem_o0, sem_o1)
        idx_v = (idx_v0, idx_v1)
        out_v = (out_v0, out_v1)

        pltpu.sync_copy(table_hbm.at[f], tab_v)

        off = lax.rem(f * 6, nch)

        def chunk_of(m):
            return lax.rem(m + off, nch)

        def start_idx(m, b):
            pltpu.async_copy(idx_hbm.at[pl.ds(chunk_of(m) * CH, CH)],
                             idx_v[b], sem_i[b])

        def wait_idx(b):
            pltpu.make_async_copy(idx_hbm.at[pl.ds(0, CH)], idx_v[b],
                                  sem_i[b]).wait()

        def compute(b):
            ib = idx_v[b]
            ob = out_v[b]

            @plsc.parallel_loop(0, CH, step=16, unroll=16)
            def _(j):
                iv = ib[pl.ds(j, 16)]
                ob[pl.ds(j, 16)] = plsc.load_gather(tab_v, [iv])

        def start_out(m, b):
            pltpu.async_copy(out_v[b],
                             out_hbm.at[f, pl.ds(chunk_of(m) * CH, CH)],
                             sem_o[b])

        def wait_out(b):
            pltpu.make_async_copy(out_v[b],
                                  out_hbm.at[f, pl.ds(0, CH)],
                                  sem_o[b]).wait()

        for b in range(2):
            start_idx(b, b)
        for b in range(2):
            wait_idx(b)
            compute(b)
            start_out(b, b)
            start_idx(b + 2, b)

        @pl.loop(2, nch - 2, step=2)
        def _(t):
            for b in range(2):
                m = t + b
                wait_idx(b)
                wait_out(b)
                compute(b)
                start_out(m, b)
                start_idx(m + 2, b)

        for b in range(2):
            wait_idx(b)
            wait_out(b)
            compute(b)
            start_out(nch - 2 + b, b)
        for b in range(2):
            wait_out(b)

    out_t = gather_kernel(idx, table_t)
    return out_t.T


# speedup vs baseline: 24.3395x; 1.3115x over previous
import functools

import jax
import jax.numpy as jnp
from jax import lax
from jax.experimental import pallas as pl
from jax.experimental.pallas import tpu as pltpu
from jax.experimental.pallas import tpu_sc as plsc

FEAT = 32
VOCAB = 100000
CH = 4096
K = 8


def kernel(element_list, encodings):
    idx = element_list.reshape(-1).astype(jnp.int32)
    B = idx.shape[0]
    table_t = encodings.T

    info = plsc.get_sparse_core_info()
    nc, ns = info.num_cores, info.num_subcores
    nch = B // CH
    nslab = nch // K

    mesh = plsc.VectorSubcoreMesh(core_axis_name="c", subcore_axis_name="s")

    @functools.partial(
        pl.kernel,
        mesh=mesh,
        out_type=jax.ShapeDtypeStruct((FEAT, B), jnp.float32),
        scratch_types=[
            pltpu.VMEM_SHARED((2, K * CH), jnp.int32),
            pltpu.VMEM((VOCAB,), jnp.float32),
            pltpu.VMEM((CH,), jnp.int32),
            pltpu.VMEM((CH,), jnp.int32),
            pltpu.VMEM((CH,), jnp.float32),
            pltpu.VMEM((CH,), jnp.float32),
            pltpu.SemaphoreType.DMA,
            pltpu.SemaphoreType.DMA,
            pltpu.SemaphoreType.DMA,
            pltpu.SemaphoreType.DMA,
            pltpu.SemaphoreType.DMA,
        ],
        compiler_params=pltpu.CompilerParams(needs_layout_passes=False),
    )
    def gather_kernel(idx_hbm, table_hbm, out_hbm, sh_idx, tab_v, idx_v0,
                      idx_v1, out_v0, out_v1, sem_i0, sem_i1, sem_o0,
                      sem_o1, sem_slab):
        c = lax.axis_index("c")
        s = lax.axis_index("s")
        f = c * ns + s
        sem_i = (sem_i0, sem_i1)
        sem_o = (sem_o0, sem_o1)
        idx_v = (idx_v0, idx_v1)
        out_v = (out_v0, out_v1)

        pltpu.sync_copy(table_hbm.at[f], tab_v)

        def slab_fetch_start(si, a):
            pltpu.async_copy(idx_hbm.at[pl.ds(si * (K * CH), K * CH)],
                             sh_idx.at[a], sem_slab)

        def slab_fetch_wait(a):
            pltpu.make_async_copy(idx_hbm.at[pl.ds(0, K * CH)],
                                  sh_idx.at[a], sem_slab).wait()

        def start_idx(kk, b, a):
            pltpu.async_copy(sh_idx.at[a, pl.ds(kk * CH, CH)], idx_v[b],
                             sem_i[b])

        def wait_idx(b):
            pltpu.make_async_copy(idx_hbm.at[pl.ds(0, CH)], idx_v[b],
                                  sem_i[b]).wait()

        def compute(b):
            ib = idx_v[b]
            ob = out_v[b]

            @plsc.parallel_loop(0, CH, step=16, unroll=16)
            def _(j):
                iv = ib[pl.ds(j, 16)]
                ob[pl.ds(j, 16)] = plsc.load_gather(tab_v, [iv])

        def start_out(m, b):
            pltpu.async_copy(out_v[b],
                             out_hbm.at[f, pl.ds(m * CH, CH)],
                             sem_o[b])

        def wait_out(b):
            pltpu.make_async_copy(out_v[b],
                                  out_hbm.at[f, pl.ds(0, CH)],
                                  sem_o[b]).wait()

        def chunk_in_slab(k):
            return lax.rem(k + s, K)

        def do_slab(si, a, prefetch, first, last):
            if prefetch:
                @pl.when(s == 0)
                def _():
                    slab_fetch_start(si + 1, 1 - a)

            kk0 = chunk_in_slab(0)
            start_idx(kk0, 0, a)
            for k in range(K):
                b = k % 2
                wait_idx(b)
                if k + 1 < K:
                    start_idx(chunk_in_slab(k + 1), 1 - b, a)
                if not first or k >= 2:
                    wait_out(b)
                compute(b)
                start_out(si * K + chunk_in_slab(k), b)

            if prefetch:
                @pl.when(s == 0)
                def _():
                    slab_fetch_wait(1 - a)

            if not last:
                plsc.subcore_barrier()

        @pl.when(s == 0)
        def _():
            slab_fetch_start(0, 0)
            slab_fetch_wait(0)

        plsc.subcore_barrier()

        do_slab(jnp.int32(0), 0, True, True, False)

        @pl.loop(1, nslab - 2, step=2)
        def _(t):
            do_slab(t, 1, True, False, False)
            do_slab(t + 1, 0, True, False, False)

        do_slab(jnp.int32(nslab - 2), 1, True, False, False)
        do_slab(jnp.int32(nslab - 1), 0, False, False, True)

        for b in range(2):
            wait_out(b)

    out_t = gather_kernel(idx, table_t)
    return out_t.T
